# baseline (speedup 1.0000x reference)
import functools

import jax
import jax.numpy as jnp
from jax import lax
from jax.experimental import pallas as pl
from jax.experimental.pallas import tpu as pltpu
from jax.experimental.pallas import tpu_sc as plsc

BATCH = 16384
NUM_FACTORS = 32
PACK = 128 // NUM_FACTORS
NUM_WORKERS = 32
PER_WORKER = BATCH // NUM_WORKERS
N_CHUNKS = 4
CHUNK = PER_WORKER // N_CHUNKS
GROUPS_PER_CHUNK = CHUNK // 16


def _sc_body(uid_hbm, iid_hbm, exp_hbm, uemb_hbm, iemb_hbm, gamma_hbm,
             bias_hbm, out_hbm,
             uid_v, iid_v, uidx_v, iidx_v, exp_v, gam_v, bias_v, out_v,
             urows0_v, urows1_v, irows0_v, irows1_v,
             gsem, usem0, usem1, isem0, isem1):
    n_cores = 2
    wid = lax.axis_index("s") * n_cores + lax.axis_index("c")
    base = wid * PER_WORKER

    pltpu.sync_copy(uid_hbm.at[pl.ds(N_CHUNKS * wid, N_CHUNKS)], uid_v)
    pltpu.sync_copy(iid_hbm.at[pl.ds(N_CHUNKS * wid, N_CHUNKS)], iid_v)
    pltpu.sync_copy(exp_hbm.at[pl.ds(base, PER_WORKER)], exp_v)
    pltpu.sync_copy(bias_hbm, bias_v)

    gamma_copies = [
        pltpu.async_copy(gamma_hbm.at[uid_v.at[j]],
                         gam_v.at[pl.ds(j * CHUNK, CHUNK)], gsem)
        for j in range(N_CHUNKS)
    ]

    for j in range(N_CHUNKS):
        for k in range(CHUNK // 16):
            sl = pl.ds(k * 16, 16)
            uidx_v[j, sl] = jax.lax.shift_right_logical(uid_v[j, sl], PACK // 2)
            iidx_v[j, sl] = jax.lax.shift_right_logical(iid_v[j, sl], PACK // 2)

    ubufs = [urows0_v, urows1_v]
    ibufs = [irows0_v, irows1_v]
    usems = [usem0, usem1]
    isems = [isem0, isem1]

    def fire(j):
        b = j % 2
        return (pltpu.async_copy(uemb_hbm.at[uidx_v.at[j]], ubufs[b], usems[b]),
                pltpu.async_copy(iemb_hbm.at[iidx_v.at[j]], ibufs[b], isems[b]))

    lane = lax.iota(jnp.int32, 16)
    bias_vec = bias_v[...]

    inflight = fire(0)
    gamma_copies[0].wait()
    gamma_copies[1].wait()
    gamma_copies[2].wait()
    gamma_copies[3].wait()

    for j in range(N_CHUNKS):
        b = j % 2
        uc, ic = inflight
        uc.wait()
        ic.wait()
        if j + 1 < N_CHUNKS:
            inflight = fire(j + 1)
        ub, ibuf = ubufs[b], ibufs[b]

        def group(g, _):
            row = g * 16 + lane
            off = j * CHUNK + g * 16
            uid16 = uid_v[j, pl.ds(g * 16, 16)]
            iid16 = iid_v[j, pl.ds(g * 16, 16)]
            ucol = (uid16 & (PACK - 1)) * NUM_FACTORS
            icol = (iid16 & (PACK - 1)) * NUM_FACTORS
            acc = gam_v[pl.ds(off, 16)] * exp_v[pl.ds(off, 16)] + bias_vec
            for d in range(NUM_FACTORS):
                u = plsc.load_gather(ub, [row, ucol + d])
                v = plsc.load_gather(ibuf, [row, icol + d])
                acc = acc + u * v
            out_v[pl.ds(off, 16)] = acc
            return _

        lax.fori_loop(0, GROUPS_PER_CHUNK, group, None)

    pltpu.sync_copy(out_v, out_hbm.at[pl.ds(base, PER_WORKER)])


@jax.jit
def kernel(user_ids, item_ids, exposures_hat, user_embeddings,
           item_embeddings, gamma, bias):
    mesh = plsc.VectorSubcoreMesh(core_axis_name="c", subcore_axis_name="s")
    uid2 = user_ids.reshape(BATCH // CHUNK, CHUNK)
    iid2 = item_ids.reshape(BATCH // CHUNK, CHUNK)
    um = user_embeddings.reshape(-1, PACK * NUM_FACTORS)
    im = item_embeddings.reshape(-1, PACK * NUM_FACTORS)
    bias16 = jnp.broadcast_to(bias, (16,))
    run = functools.partial(
        pl.kernel,
        mesh=mesh,
        compiler_params=pltpu.CompilerParams(
            needs_layout_passes=False, use_tc_tiling_on_sc=True),
        out_type=jax.ShapeDtypeStruct((BATCH,), jnp.float32),
        scratch_types=[
            pltpu.VMEM((N_CHUNKS, CHUNK), jnp.int32),
            pltpu.VMEM((N_CHUNKS, CHUNK), jnp.int32),
            pltpu.VMEM((N_CHUNKS, CHUNK), jnp.int32),
            pltpu.VMEM((N_CHUNKS, CHUNK), jnp.int32),
            pltpu.VMEM((PER_WORKER,), jnp.float32),
            pltpu.VMEM((PER_WORKER,), jnp.float32),
            pltpu.VMEM((16,), jnp.float32),
            pltpu.VMEM((PER_WORKER,), jnp.float32),
            pltpu.VMEM((CHUNK, PACK * NUM_FACTORS), jnp.float32),
            pltpu.VMEM((CHUNK, PACK * NUM_FACTORS), jnp.float32),
            pltpu.VMEM((CHUNK, PACK * NUM_FACTORS), jnp.float32),
            pltpu.VMEM((CHUNK, PACK * NUM_FACTORS), jnp.float32),
            pltpu.SemaphoreType.DMA,
            pltpu.SemaphoreType.DMA,
            pltpu.SemaphoreType.DMA,
            pltpu.SemaphoreType.DMA,
            pltpu.SemaphoreType.DMA,
        ],
    )(_sc_body)
    return run(uid2, iid2, exposures_hat, um, im, gamma, bias16)
